# baseline (speedup 1.0000x reference)
import functools

import numpy as _np

import jax
import jax.numpy as jnp
from jax import lax
from jax.experimental import pallas as pl
from jax.experimental.pallas import tpu as pltpu
from jax.experimental.pallas import tpu_sc as plsc

_N = 10000
_E = 160000
_D = 256
_HID = 512
_HALF = _D // 2

_NC = 2
_NS = 16
_NPAD = 10112
_EPT1 = _E // (_NC * _NS)
_EPT = _E // _NS
_CH = 80
_NCH = _EPT // _CH
_RPT = _NPAD // _NS

_RB = 1280
_GRID = -(-_N // _RB)

_sc_mesh = plsc.VectorSubcoreMesh(core_axis_name="c", subcore_axis_name="s")

_CH1 = 80
_NCH1 = _EPT1 // _CH1
_TAIL1 = _EPT1 + 8 - _NCH1 * _CH1
_DW = 16


@functools.partial(
    pl.kernel,
    out_type=jax.ShapeDtypeStruct((_NC, _NPAD, _DW), jnp.float32),
    mesh=_sc_mesh,
    scratch_types=[
        pltpu.VMEM((_EPT1 + 16,), jnp.int32),
        pltpu.VMEM((_EPT1 + 16,), jnp.float32),
        pltpu.VMEM((_CH1,), jnp.int32),
        pltpu.VMEM((_TAIL1,), jnp.int32),
        pltpu.VMEM((_CH1, _DW), jnp.float32),
        pltpu.VMEM((128, _DW), jnp.float32),
        pltpu.VMEM_SHARED((_NPAD, _DW), jnp.float32),
    ],
)
def _deg_kernel(dst_hbm, ew_hbm, out_hbm, dst_v, ew_v, didx, didx_t, buf, zb,
                acc_sh):
    cid = lax.axis_index("c")
    sid = lax.axis_index("s")
    base = (cid * _NS + sid) * _EPT1

    zeros16 = jnp.zeros((_DW,), jnp.float32)
    lanes = lax.iota(jnp.int32, 16)

    def zzb(i, _):
        zb[i, pl.ds(0, _DW)] = zeros16
        return 0

    lax.fori_loop(0, 128, zzb, 0)

    rbase = sid * _RPT
    for k in range(4):
        pltpu.sync_copy(zb, acc_sh.at[pl.ds(rbase + 128 * k, 128)])
    pltpu.sync_copy(zb.at[pl.ds(0, _RPT - 512)],
                    acc_sh.at[pl.ds(rbase + 512, _RPT - 512)])

    pltpu.sync_copy(dst_hbm.at[pl.ds(base, _EPT1)], dst_v.at[pl.ds(0, _EPT1)])
    pltpu.sync_copy(ew_hbm.at[pl.ds(base, _EPT1)], ew_v.at[pl.ds(0, _EPT1)])

    keep = lanes < 8
    tail = _EPT1 - 8
    dst_v[pl.ds(tail, 16)] = jnp.where(keep, dst_v[pl.ds(tail, 16)], 0)
    ew_v[pl.ds(tail, 16)] = jnp.where(keep, ew_v[pl.ds(tail, 16)], 0.0)

    plsc.subcore_barrier()

    def build(g, nrow, dbuf):
        for j in range(nrow // 16):
            dbuf[pl.ds(16 * j, 16)] = dst_v[pl.ds(g * _CH1 + 16 * j, 16)]
        for k in range(nrow // 16):
            wv = ew_v[pl.ds(g * _CH1 + 16 * k, 16)]
            for r in range(16):
                buf[16 * k + r, pl.ds(0, _DW)] = jnp.where(lanes == r, wv, 0.0)

    def chunk(g, _):
        build(g, _CH1, didx)
        pltpu.sync_copy(buf, acc_sh.at[didx], add=True)
        return 0

    lax.fori_loop(0, _NCH1, chunk, 0)
    build(_NCH1, _TAIL1, didx_t)
    pltpu.sync_copy(buf.at[pl.ds(0, _TAIL1)], acc_sh.at[didx_t], add=True)

    plsc.subcore_barrier()
    pltpu.sync_copy(acc_sh.at[pl.ds(rbase, _RPT)],
                    out_hbm.at[cid, pl.ds(rbase, _RPT)])


@functools.partial(
    pl.kernel,
    out_type=jax.ShapeDtypeStruct((_NC, _NPAD, _HALF), jnp.float32),
    mesh=_sc_mesh,
    scratch_types=[
        pltpu.VMEM((_EPT,), jnp.int32),
        pltpu.VMEM((_CH,), jnp.int32),
        pltpu.VMEM((_CH,), jnp.int32),
        pltpu.VMEM((_CH,), jnp.int32),
        pltpu.VMEM((_CH,), jnp.int32),
        pltpu.VMEM((_CH,), jnp.int32),
        pltpu.VMEM((_CH,), jnp.int32),
        pltpu.VMEM((_CH,), jnp.float32),
        pltpu.VMEM((_CH,), jnp.float32),
        pltpu.VMEM((_CH,), jnp.float32),
        pltpu.VMEM((_CH, _HALF), jnp.float32),
        pltpu.VMEM((_CH, _HALF), jnp.float32),
        pltpu.VMEM((_CH, _HALF), jnp.float32),
        pltpu.VMEM_SHARED((_NPAD, _HALF), jnp.float32),
        pltpu.SemaphoreType.DMA,
        pltpu.SemaphoreType.DMA,
        pltpu.SemaphoreType.DMA,
        pltpu.SemaphoreType.DMA,
        pltpu.SemaphoreType.DMA,
        pltpu.SemaphoreType.DMA,
        pltpu.SemaphoreType.DMA,
        pltpu.SemaphoreType.DMA,
        pltpu.SemaphoreType.DMA,
    ],
)
def _msg_kernel(y_hbm, src_hbm, dst_hbm, ew_hbm, out_hbm,
                src_v, sidx0, sidx1, sidx2, didx0, didx1, didx2,
                ewb0, ewb1, ewb2, buf0, buf1, buf2, acc_sh,
                semg0, semg1, semg2, seme0, seme1, seme2,
                sems0, sems1, sems2):
    cid = lax.axis_index("c")
    sid = lax.axis_index("s")
    base = sid * _EPT
    coff = cid * _N

    slots = (
        (sidx0, didx0, ewb0, buf0, semg0, seme0, sems0),
        (sidx1, didx1, ewb1, buf1, semg1, seme1, sems1),
        (sidx2, didx2, ewb2, buf2, semg2, seme2, sems2),
    )

    pltpu.sync_copy(src_hbm.at[pl.ds(base, _EPT)], src_v)

    zeros16 = jnp.zeros((16,), jnp.float32)

    def zb(i, _):
        for j in range(_HALF // 16):
            buf0[i, pl.ds(16 * j, 16)] = zeros16
        return 0

    lax.fori_loop(0, _CH, zb, 0)

    rbase = sid * _RPT
    nfull = _RPT // _CH
    for k in range(nfull):
        pltpu.sync_copy(buf0.at[pl.ds(0, _CH)],
                        acc_sh.at[pl.ds(rbase + _CH * k, _CH)])
    rem = _RPT - nfull * _CH
    pltpu.sync_copy(buf0.at[pl.ds(0, rem)],
                    acc_sh.at[pl.ds(rbase + nfull * _CH, rem)])

    plsc.subcore_barrier()

    def fire(g, slot):
        sbuf, dbuf, ebuf, gbuf, mg, me, _ = slot
        for j in range(_CH // 16):
            sbuf[pl.ds(16 * j, 16)] = src_v[pl.ds(g * _CH + 16 * j, 16)] + coff
        pltpu.async_copy(y_hbm.at[sbuf], gbuf, mg)
        pltpu.async_copy(dst_hbm.at[pl.ds(base + g * _CH, _CH)], dbuf, me)
        pltpu.async_copy(ew_hbm.at[pl.ds(base + g * _CH, _CH)], ebuf, me)

    def step(g, slot, slot_next):
        sA, dA, eA, bA, mgA, meA, msA = slot

        @pl.when(g + 2 < _NCH)
        def _():
            sN, dN, eN, bN, _mgN, _meN, msN = slot_next

            @pl.when(g >= 1)
            def _():
                pltpu.make_async_copy(bN, acc_sh.at[dN], msN).wait()

            fire(g + 2, slot_next)

        pltpu.make_async_copy(y_hbm.at[sA], bA, mgA).wait()
        pltpu.make_async_copy(dst_hbm.at[pl.ds(base + g * _CH, _CH)], dA,
                              meA).wait()
        pltpu.make_async_copy(ew_hbm.at[pl.ds(base + g * _CH, _CH)], eA,
                              meA).wait()

        for k in range(_CH // 16):
            wv = eA[pl.ds(16 * k, 16)]
            for r in range(16):
                w = jnp.take(wv, jnp.full((16,), r, jnp.int32))
                e = 16 * k + r
                for j in range(_HALF // 16):
                    bA[e, pl.ds(16 * j, 16)] = bA[e, pl.ds(16 * j, 16)] * w

        pltpu.async_copy(bA, acc_sh.at[dA], msA, add=True)

    fire(0, slots[0])
    fire(1, slots[1])

    def loop_body(g, _):
        for p in range(3):
            @pl.when(g % 3 == p)
            def _():
                step(g, slots[p], slots[(p + 2) % 3])

        return 0

    lax.fori_loop(0, _NCH, loop_body, 0)

    for p in range(3):
        gg = _NCH - 3 + p
        sP, dP, eP, bP, _mg, _me, msP = slots[gg % 3]
        pltpu.make_async_copy(bP, acc_sh.at[dP], msP).wait()
    plsc.subcore_barrier()
    pltpu.sync_copy(acc_sh.at[pl.ds(rbase, _RPT)],
                    out_hbm.at[cid, pl.ds(rbase, _RPT)])


def _fold_body(wz_ref, lzt_ref, wh_ref, lht_ref,
               bz_ref, lzb_ref, bh_ref, lhb_ref, wc_ref, bc_ref):
    wc_ref[:, :_HID] = jnp.dot(wz_ref[...], lzt_ref[...],
                               preferred_element_type=jnp.float32)
    wc_ref[:, _HID:] = jnp.dot(wh_ref[...], lht_ref[...],
                               preferred_element_type=jnp.float32)
    bc_ref[:, :_HID] = jnp.dot(bz_ref[...], lzt_ref[...],
                               preferred_element_type=jnp.float32) + lzb_ref[...]
    bc_ref[:, _HID:] = jnp.dot(bh_ref[...], lht_ref[...],
                               preferred_element_type=jnp.float32) + lhb_ref[...]


_fold_kernel = pl.pallas_call(
    _fold_body,
    out_shape=(
        jax.ShapeDtypeStruct((_D, 2 * _HID), jnp.float32),
        jax.ShapeDtypeStruct((1, 2 * _HID), jnp.float32),
    ),
)


def _y_body(x_ref, degw_ref, y2_ref):
    d = degw_ref[...]
    deg = jnp.sum(d[0] + d[1], axis=1, keepdims=True) + 1.0
    dis = lax.rsqrt(deg)
    y = x_ref[...] * dis
    y2_ref[0] = y[:, :_HALF]
    y2_ref[1] = y[:, _HALF:]


_y_kernel = pl.pallas_call(
    _y_body,
    grid=(_GRID,),
    in_specs=[
        pl.BlockSpec((_RB, _D), lambda i: (i, 0)),
        pl.BlockSpec((_NC, _RB, _DW), lambda i: (0, i, 0)),
    ],
    out_specs=pl.BlockSpec((_NC, _RB, _HALF), lambda i: (0, i, 0)),
    out_shape=jax.ShapeDtypeStruct((_NC, _N, _HALF), jnp.float32),
)


def _head_body(sy2_ref, degw_ref, x_ref, wc_ref, bc_ref, lwt_ref, lb_ref,
               out_ref):
    d = degw_ref[...]
    deg = jnp.sum(d[0] + d[1], axis=1, keepdims=True) + 1.0
    dis = lax.rsqrt(deg)
    sy = jnp.concatenate([sy2_ref[0], sy2_ref[1]], axis=1)
    a = dis * sy + (dis * dis) * x_ref[...]
    zh = jnp.dot(a, wc_ref[...], preferred_element_type=jnp.float32) + bc_ref[...]
    z = jax.nn.sigmoid(zh[:, :_HID])
    ht = jnp.tanh(zh[:, _HID:])
    g = (1.0 - z) * ht * lwt_ref[...]
    out_ref[...] = jnp.sum(g, axis=1, keepdims=True) + lb_ref[...]


_head_kernel = pl.pallas_call(
    _head_body,
    grid=(_GRID,),
    in_specs=[
        pl.BlockSpec((_NC, _RB, _HALF), lambda i: (0, i, 0)),
        pl.BlockSpec((_NC, _RB, _DW), lambda i: (0, i, 0)),
        pl.BlockSpec((_RB, _D), lambda i: (i, 0)),
        pl.BlockSpec((_D, 2 * _HID), lambda i: (0, 0)),
        pl.BlockSpec((1, 2 * _HID), lambda i: (0, 0)),
        pl.BlockSpec((1, _HID), lambda i: (0, 0)),
        pl.BlockSpec((1, 1), lambda i: (0, 0)),
    ],
    out_specs=pl.BlockSpec((_RB, 1), lambda i: (i, 0)),
    out_shape=jax.ShapeDtypeStruct((_N, 1), jnp.float32),
)


def kernel(x, edge_index, edge_weight, Wz, bz, Wr, br, Wh, bh,
           LzW, Lzb, LrW, Lrb, LhW, Lhb, linW, linb):
    src = edge_index[0]
    dst = edge_index[1]
    degw = _deg_kernel(dst, edge_weight)
    wc, bc = _fold_kernel(Wz, LzW[:_HID], Wh, LhW[:_HID],
                          bz.reshape(1, _HID), Lzb.reshape(1, _HID),
                          bh.reshape(1, _HID), Lhb.reshape(1, _HID))
    y2 = _y_kernel(x, degw)
    yflat = y2.reshape(_NC * _N, _HALF)
    sy2 = _msg_kernel(yflat, src, dst, edge_weight)
    out = _head_kernel(sy2, degw, x, wc, bc,
                       linW.reshape(1, _HID), linb.reshape(1, 1))
    return out.reshape(_N)


# speedup vs baseline: 1.2552x; 1.2552x over previous
import functools

import numpy as _np

import jax
import jax.numpy as jnp
from jax import lax
from jax.experimental import pallas as pl
from jax.experimental.pallas import tpu as pltpu
from jax.experimental.pallas import tpu_sc as plsc

_N = 10000
_E = 160000
_D = 256
_HID = 512
_HALF = _D // 2

_NC = 2
_NS = 16
_NPAD = 10112
_EPT1 = _E // (_NC * _NS)
_EPT = _E // _NS
_CH = 80
_NCH = _EPT // _CH
_RPT = _NPAD // _NS

_RB = 1280
_GRID = -(-_N // _RB)

_sc_mesh = plsc.VectorSubcoreMesh(core_axis_name="c", subcore_axis_name="s")

_CH1 = 80
_NCH1 = _EPT1 // _CH1
_TAIL1 = _EPT1 + 8 - _NCH1 * _CH1
_DW = 16


@functools.partial(
    pl.kernel,
    out_type=jax.ShapeDtypeStruct((_NC, _NPAD, _DW), jnp.float32),
    mesh=_sc_mesh,
    scratch_types=[
        pltpu.VMEM((_EPT1 + 16,), jnp.int32),
        pltpu.VMEM((_EPT1 + 16,), jnp.float32),
        pltpu.VMEM((_CH1,), jnp.int32),
        pltpu.VMEM((_TAIL1,), jnp.int32),
        pltpu.VMEM((_CH1, _DW), jnp.float32),
        pltpu.VMEM((128, _DW), jnp.float32),
        pltpu.VMEM_SHARED((_NPAD, _DW), jnp.float32),
    ],
)
def _deg_kernel(dst_hbm, ew_hbm, out_hbm, dst_v, ew_v, didx, didx_t, buf, zb,
                acc_sh):
    cid = lax.axis_index("c")
    sid = lax.axis_index("s")
    base = (cid * _NS + sid) * _EPT1

    zeros16 = jnp.zeros((_DW,), jnp.float32)
    lanes = lax.iota(jnp.int32, 16)

    def zzb(i, _):
        zb[i, pl.ds(0, _DW)] = zeros16
        return 0

    lax.fori_loop(0, 128, zzb, 0)

    rbase = sid * _RPT
    for k in range(4):
        pltpu.sync_copy(zb, acc_sh.at[pl.ds(rbase + 128 * k, 128)])
    pltpu.sync_copy(zb.at[pl.ds(0, _RPT - 512)],
                    acc_sh.at[pl.ds(rbase + 512, _RPT - 512)])

    pltpu.sync_copy(dst_hbm.at[pl.ds(base, _EPT1)], dst_v.at[pl.ds(0, _EPT1)])
    pltpu.sync_copy(ew_hbm.at[pl.ds(base, _EPT1)], ew_v.at[pl.ds(0, _EPT1)])

    keep = lanes < 8
    tail = _EPT1 - 8
    dst_v[pl.ds(tail, 16)] = jnp.where(keep, dst_v[pl.ds(tail, 16)], 0)
    ew_v[pl.ds(tail, 16)] = jnp.where(keep, ew_v[pl.ds(tail, 16)], 0.0)

    plsc.subcore_barrier()

    def build(g, nrow, dbuf):
        for j in range(nrow // 16):
            dbuf[pl.ds(16 * j, 16)] = dst_v[pl.ds(g * _CH1 + 16 * j, 16)]
        for k in range(nrow // 16):
            wv = ew_v[pl.ds(g * _CH1 + 16 * k, 16)]
            for r in range(16):
                buf[16 * k + r, pl.ds(0, _DW)] = jnp.where(lanes == r, wv, 0.0)

    def chunk(g, _):
        build(g, _CH1, didx)
        pltpu.sync_copy(buf, acc_sh.at[didx], add=True)
        return 0

    lax.fori_loop(0, _NCH1, chunk, 0)
    build(_NCH1, _TAIL1, didx_t)
    pltpu.sync_copy(buf.at[pl.ds(0, _TAIL1)], acc_sh.at[didx_t], add=True)

    plsc.subcore_barrier()
    pltpu.sync_copy(acc_sh.at[pl.ds(rbase, _RPT)],
                    out_hbm.at[cid, pl.ds(rbase, _RPT)])


@functools.partial(
    pl.kernel,
    out_type=jax.ShapeDtypeStruct((_NC, _NPAD, _HALF), jnp.float32),
    mesh=_sc_mesh,
    scratch_types=[
        pltpu.VMEM((_EPT,), jnp.int32),
        pltpu.VMEM((_CH,), jnp.int32),
        pltpu.VMEM((_CH,), jnp.int32),
        pltpu.VMEM((_CH,), jnp.int32),
        pltpu.VMEM((_CH,), jnp.int32),
        pltpu.VMEM((_CH,), jnp.float32),
        pltpu.VMEM((_CH,), jnp.float32),
        pltpu.VMEM((_CH, _HALF), jnp.float32),
        pltpu.VMEM((_CH, _HALF), jnp.float32),
        pltpu.VMEM_SHARED((_NPAD, _HALF), jnp.float32),
        pltpu.SemaphoreType.DMA,
        pltpu.SemaphoreType.DMA,
        pltpu.SemaphoreType.DMA,
        pltpu.SemaphoreType.DMA,
        pltpu.SemaphoreType.DMA,
        pltpu.SemaphoreType.DMA,
    ],
)
def _msg_kernel(y_hbm, src_hbm, dst_hbm, ew_hbm, out_hbm,
                src_v, sidx0, sidx1, didx0, didx1, ewb0, ewb1,
                buf0, buf1, acc_sh, semg0, semg1, seme0, seme1,
                sems0, sems1):
    cid = lax.axis_index("c")
    sid = lax.axis_index("s")
    base = sid * _EPT
    coff = cid * _N

    pltpu.sync_copy(src_hbm.at[pl.ds(base, _EPT)], src_v)

    zeros16 = jnp.zeros((16,), jnp.float32)

    def zb(i, _):
        for j in range(_HALF // 16):
            buf0[i, pl.ds(16 * j, 16)] = zeros16
        return 0

    lax.fori_loop(0, _CH, zb, 0)

    rbase = sid * _RPT
    nfull = _RPT // _CH
    for k in range(nfull):
        pltpu.sync_copy(buf0.at[pl.ds(0, _CH)],
                        acc_sh.at[pl.ds(rbase + _CH * k, _CH)])
    rem = _RPT - nfull * _CH
    pltpu.sync_copy(buf0.at[pl.ds(0, rem)],
                    acc_sh.at[pl.ds(rbase + nfull * _CH, rem)])

    plsc.subcore_barrier()

    def fire(g, sbuf, dbuf, ebuf, gbuf, mg, me):
        for j in range(_CH // 16):
            sbuf[pl.ds(16 * j, 16)] = src_v[pl.ds(g * _CH + 16 * j, 16)] + coff
        pltpu.async_copy(y_hbm.at[sbuf], gbuf, mg)
        pltpu.async_copy(dst_hbm.at[pl.ds(base + g * _CH, _CH)], dbuf, me)
        pltpu.async_copy(ew_hbm.at[pl.ds(base + g * _CH, _CH)], ebuf, me)

    def step(g, sA, dA, eA, bA, mgA, meA, msA,
             sB, dB, eB, bB, mgB, meB, msB):
        @pl.when(g + 1 < _NCH)
        def _():
            fire(g + 1, sB, dB, eB, bB, mgB, meB)

        pltpu.make_async_copy(y_hbm.at[sA], bA, mgA).wait()
        pltpu.make_async_copy(dst_hbm.at[pl.ds(base + g * _CH, _CH)], dA,
                              meA).wait()
        pltpu.make_async_copy(ew_hbm.at[pl.ds(base + g * _CH, _CH)], eA,
                              meA).wait()

        def scale(k, _):
            wv = eA[pl.ds(16 * k, 16)]
            for r in range(16):
                w = jnp.take(wv, jnp.full((16,), r, jnp.int32))
                e = 16 * k + r
                for j in range(_HALF // 16):
                    bA[e, pl.ds(16 * j, 16)] = bA[e, pl.ds(16 * j, 16)] * w
            return 0

        lax.fori_loop(0, _CH // 16, scale, 0)
        pltpu.sync_copy(bA, acc_sh.at[dA], add=True)

    fire(0, sidx0, didx0, ewb0, buf0, semg0, seme0)

    def loop_body(g, _):
        @pl.when(g % 2 == 0)
        def _():
            step(g, sidx0, didx0, ewb0, buf0, semg0, seme0, sems0,
                 sidx1, didx1, ewb1, buf1, semg1, seme1, sems1)

        @pl.when(g % 2 == 1)
        def _():
            step(g, sidx1, didx1, ewb1, buf1, semg1, seme1, sems1,
                 sidx0, didx0, ewb0, buf0, semg0, seme0, sems0)

        return 0

    lax.fori_loop(0, _NCH, loop_body, 0)

    plsc.subcore_barrier()
    pltpu.sync_copy(acc_sh.at[pl.ds(rbase, _RPT)],
                    out_hbm.at[cid, pl.ds(rbase, _RPT)])


def _fold_body(wz_ref, lzt_ref, wh_ref, lht_ref,
               bz_ref, lzb_ref, bh_ref, lhb_ref, wc_ref, bc_ref):
    wc_ref[:, :_HID] = jnp.dot(wz_ref[...], lzt_ref[...],
                               preferred_element_type=jnp.float32)
    wc_ref[:, _HID:] = jnp.dot(wh_ref[...], lht_ref[...],
                               preferred_element_type=jnp.float32)
    bc_ref[:, :_HID] = jnp.dot(bz_ref[...], lzt_ref[...],
                               preferred_element_type=jnp.float32) + lzb_ref[...]
    bc_ref[:, _HID:] = jnp.dot(bh_ref[...], lht_ref[...],
                               preferred_element_type=jnp.float32) + lhb_ref[...]


_fold_kernel = pl.pallas_call(
    _fold_body,
    out_shape=(
        jax.ShapeDtypeStruct((_D, 2 * _HID), jnp.float32),
        jax.ShapeDtypeStruct((1, 2 * _HID), jnp.float32),
    ),
)


def _y_body(x_ref, degw_ref, y2_ref):
    d = degw_ref[...]
    deg = jnp.sum(d[0] + d[1], axis=1, keepdims=True) + 1.0
    dis = lax.rsqrt(deg)
    y = x_ref[...] * dis
    y2_ref[0] = y[:, :_HALF]
    y2_ref[1] = y[:, _HALF:]


_y_kernel = pl.pallas_call(
    _y_body,
    grid=(_GRID,),
    in_specs=[
        pl.BlockSpec((_RB, _D), lambda i: (i, 0)),
        pl.BlockSpec((_NC, _RB, _DW), lambda i: (0, i, 0)),
    ],
    out_specs=pl.BlockSpec((_NC, _RB, _HALF), lambda i: (0, i, 0)),
    out_shape=jax.ShapeDtypeStruct((_NC, _N, _HALF), jnp.float32),
)


def _head_body(sy2_ref, degw_ref, x_ref, wc_ref, bc_ref, lwt_ref, lb_ref,
               out_ref):
    d = degw_ref[...]
    deg = jnp.sum(d[0] + d[1], axis=1, keepdims=True) + 1.0
    dis = lax.rsqrt(deg)
    sy = jnp.concatenate([sy2_ref[0], sy2_ref[1]], axis=1)
    a = dis * sy + (dis * dis) * x_ref[...]
    zh = jnp.dot(a, wc_ref[...], preferred_element_type=jnp.float32) + bc_ref[...]
    z = jax.nn.sigmoid(zh[:, :_HID])
    ht = jnp.tanh(zh[:, _HID:])
    g = (1.0 - z) * ht * lwt_ref[...]
    out_ref[...] = jnp.sum(g, axis=1, keepdims=True) + lb_ref[...]


_head_kernel = pl.pallas_call(
    _head_body,
    grid=(_GRID,),
    in_specs=[
        pl.BlockSpec((_NC, _RB, _HALF), lambda i: (0, i, 0)),
        pl.BlockSpec((_NC, _RB, _DW), lambda i: (0, i, 0)),
        pl.BlockSpec((_RB, _D), lambda i: (i, 0)),
        pl.BlockSpec((_D, 2 * _HID), lambda i: (0, 0)),
        pl.BlockSpec((1, 2 * _HID), lambda i: (0, 0)),
        pl.BlockSpec((1, _HID), lambda i: (0, 0)),
        pl.BlockSpec((1, 1), lambda i: (0, 0)),
    ],
    out_specs=pl.BlockSpec((_RB, 1), lambda i: (i, 0)),
    out_shape=jax.ShapeDtypeStruct((_N, 1), jnp.float32),
)


def kernel(x, edge_index, edge_weight, Wz, bz, Wr, br, Wh, bh,
           LzW, Lzb, LrW, Lrb, LhW, Lhb, linW, linb):
    src = edge_index[0]
    dst = edge_index[1]
    degw = _deg_kernel(dst, edge_weight)
    wc, bc = _fold_kernel(Wz, LzW[:_HID], Wh, LhW[:_HID],
                          bz.reshape(1, _HID), Lzb.reshape(1, _HID),
                          bh.reshape(1, _HID), Lhb.reshape(1, _HID))
    y2 = _y_kernel(x, degw)
    yflat = y2.reshape(_NC * _N, _HALF)
    sy2 = _msg_kernel(yflat, src, dst, edge_weight)
    out = _head_kernel(sy2, degw, x, wc, bc,
                       linW.reshape(1, _HID), linb.reshape(1, 1))
    return out.reshape(_N)


# speedup vs baseline: 1.2579x; 1.0022x over previous
import functools

import numpy as _np

import jax
import jax.numpy as jnp
from jax import lax
from jax.experimental import pallas as pl
from jax.experimental.pallas import tpu as pltpu
from jax.experimental.pallas import tpu_sc as plsc

_N = 10000
_E = 160000
_D = 256
_HID = 512
_HALF = _D // 2

_NC = 2
_NS = 16
_NPAD = 10112
_EPT1 = _E // (_NC * _NS)
_EPT = _E // _NS
_CH = 80
_NCH = _EPT // _CH
_RPT = _NPAD // _NS

_RB = 1280
_GRID = -(-_N // _RB)

_sc_mesh = plsc.VectorSubcoreMesh(core_axis_name="c", subcore_axis_name="s")

_CH1 = 80
_NCH1 = _EPT1 // _CH1
_TAIL1 = _EPT1 + 8 - _NCH1 * _CH1
_DW = 16


@functools.partial(
    pl.kernel,
    out_type=jax.ShapeDtypeStruct((_NC, _NPAD, _DW), jnp.float32),
    mesh=_sc_mesh,
    scratch_types=[
        pltpu.VMEM((_EPT1 + 16,), jnp.int32),
        pltpu.VMEM((_EPT1 + 16,), jnp.float32),
        pltpu.VMEM((_CH1,), jnp.int32),
        pltpu.VMEM((_TAIL1,), jnp.int32),
        pltpu.VMEM((_CH1, _DW), jnp.float32),
        pltpu.VMEM((128, _DW), jnp.float32),
        pltpu.VMEM_SHARED((_NPAD, _DW), jnp.float32),
    ],
)
def _deg_kernel(dst_hbm, ew_hbm, out_hbm, dst_v, ew_v, didx, didx_t, buf, zb,
                acc_sh):
    cid = lax.axis_index("c")
    sid = lax.axis_index("s")
    base = (cid * _NS + sid) * _EPT1

    zeros16 = jnp.zeros((_DW,), jnp.float32)
    lanes = lax.iota(jnp.int32, 16)

    def zzb(i, _):
        zb[i, pl.ds(0, _DW)] = zeros16
        return 0

    lax.fori_loop(0, 128, zzb, 0)

    rbase = sid * _RPT
    for k in range(4):
        pltpu.sync_copy(zb, acc_sh.at[pl.ds(rbase + 128 * k, 128)])
    pltpu.sync_copy(zb.at[pl.ds(0, _RPT - 512)],
                    acc_sh.at[pl.ds(rbase + 512, _RPT - 512)])

    pltpu.sync_copy(dst_hbm.at[pl.ds(base, _EPT1)], dst_v.at[pl.ds(0, _EPT1)])
    pltpu.sync_copy(ew_hbm.at[pl.ds(base, _EPT1)], ew_v.at[pl.ds(0, _EPT1)])

    keep = lanes < 8
    tail = _EPT1 - 8
    dst_v[pl.ds(tail, 16)] = jnp.where(keep, dst_v[pl.ds(tail, 16)], 0)
    ew_v[pl.ds(tail, 16)] = jnp.where(keep, ew_v[pl.ds(tail, 16)], 0.0)

    plsc.subcore_barrier()

    def build(g, nrow, dbuf):
        for j in range(nrow // 16):
            dbuf[pl.ds(16 * j, 16)] = dst_v[pl.ds(g * _CH1 + 16 * j, 16)]
        for k in range(nrow // 16):
            wv = ew_v[pl.ds(g * _CH1 + 16 * k, 16)]
            for r in range(16):
                buf[16 * k + r, pl.ds(0, _DW)] = jnp.where(lanes == r, wv, 0.0)

    def chunk(g, _):
        build(g, _CH1, didx)
        pltpu.sync_copy(buf, acc_sh.at[didx], add=True)
        return 0

    lax.fori_loop(0, _NCH1, chunk, 0)
    build(_NCH1, _TAIL1, didx_t)
    pltpu.sync_copy(buf.at[pl.ds(0, _TAIL1)], acc_sh.at[didx_t], add=True)

    plsc.subcore_barrier()
    pltpu.sync_copy(acc_sh.at[pl.ds(rbase, _RPT)],
                    out_hbm.at[cid, pl.ds(rbase, _RPT)])


@functools.partial(
    pl.kernel,
    out_type=jax.ShapeDtypeStruct((_NC, _NPAD, _HALF), jnp.float32),
    mesh=_sc_mesh,
    scratch_types=[
        pltpu.VMEM((_EPT,), jnp.int32),
        pltpu.VMEM((_CH,), jnp.int32),
        pltpu.VMEM((_CH,), jnp.int32),
        pltpu.VMEM((_CH,), jnp.int32),
        pltpu.VMEM((_CH,), jnp.int32),
        pltpu.VMEM((_CH,), jnp.float32),
        pltpu.VMEM((_CH,), jnp.float32),
        pltpu.VMEM((_CH, _HALF), jnp.float32),
        pltpu.VMEM((_CH, _HALF), jnp.float32),
        pltpu.VMEM_SHARED((_NPAD, _HALF), jnp.float32),
        pltpu.SemaphoreType.DMA,
        pltpu.SemaphoreType.DMA,
        pltpu.SemaphoreType.DMA,
        pltpu.SemaphoreType.DMA,
    ],
)
def _msg_kernel(y_hbm, src_hbm, dst_hbm, ew_hbm, out_hbm,
                src_v, sidx0, sidx1, didx0, didx1, ewb0, ewb1,
                buf0, buf1, acc_sh, semg0, semg1, seme0, seme1):
    cid = lax.axis_index("c")
    sid = lax.axis_index("s")
    base = sid * _EPT
    coff = cid * _N

    pltpu.sync_copy(src_hbm.at[pl.ds(base, _EPT)], src_v)

    zeros16 = jnp.zeros((16,), jnp.float32)

    def zb(i, _):
        for j in range(_HALF // 16):
            buf0[i, pl.ds(16 * j, 16)] = zeros16
        return 0

    lax.fori_loop(0, _CH, zb, 0)

    rbase = sid * _RPT
    nfull = _RPT // _CH
    for k in range(nfull):
        pltpu.sync_copy(buf0.at[pl.ds(0, _CH)],
                        acc_sh.at[pl.ds(rbase + _CH * k, _CH)])
    rem = _RPT - nfull * _CH
    pltpu.sync_copy(buf0.at[pl.ds(0, rem)],
                    acc_sh.at[pl.ds(rbase + nfull * _CH, rem)])

    plsc.subcore_barrier()

    def fire(g, sbuf, dbuf, ebuf, gbuf, mg, me):
        for j in range(_CH // 16):
            sbuf[pl.ds(16 * j, 16)] = src_v[pl.ds(g * _CH + 16 * j, 16)] + coff
        pltpu.async_copy(y_hbm.at[sbuf], gbuf, mg)
        pltpu.async_copy(dst_hbm.at[pl.ds(base + g * _CH, _CH)], dbuf, me)
        pltpu.async_copy(ew_hbm.at[pl.ds(base + g * _CH, _CH)], ebuf, me)

    def step(g, sA, dA, eA, bA, mgA, meA, sB, dB, eB, bB, mgB, meB):
        @pl.when(g + 1 < _NCH)
        def _():
            fire(g + 1, sB, dB, eB, bB, mgB, meB)

        pltpu.make_async_copy(y_hbm.at[sA], bA, mgA).wait()
        pltpu.make_async_copy(dst_hbm.at[pl.ds(base + g * _CH, _CH)], dA,
                              meA).wait()
        pltpu.make_async_copy(ew_hbm.at[pl.ds(base + g * _CH, _CH)], eA,
                              meA).wait()

        def scale(k, _):
            wv = eA[pl.ds(16 * k, 16)]
            for r in range(16):
                w = jnp.take(wv, jnp.full((16,), r, jnp.int32))
                e = 16 * k + r
                for j in range(_HALF // 16):
                    bA[e, pl.ds(16 * j, 16)] = bA[e, pl.ds(16 * j, 16)] * w
            return 0

        lax.fori_loop(0, _CH // 16, scale, 0)
        pltpu.sync_copy(bA, acc_sh.at[dA], add=True)

    fire(0, sidx0, didx0, ewb0, buf0, semg0, seme0)

    def loop_body(g, _):
        @pl.when(g % 2 == 0)
        def _():
            step(g, sidx0, didx0, ewb0, buf0, semg0, seme0,
                 sidx1, didx1, ewb1, buf1, semg1, seme1)

        @pl.when(g % 2 == 1)
        def _():
            step(g, sidx1, didx1, ewb1, buf1, semg1, seme1,
                 sidx0, didx0, ewb0, buf0, semg0, seme0)

        return 0

    lax.fori_loop(0, _NCH, loop_body, 0)

    plsc.subcore_barrier()
    pltpu.sync_copy(acc_sh.at[pl.ds(rbase, _RPT)],
                    out_hbm.at[cid, pl.ds(rbase, _RPT)])


def _fold_body(wz_ref, lzt_ref, wh_ref, lht_ref,
               bz_ref, lzb_ref, bh_ref, lhb_ref, wc_ref, bc_ref):
    wc_ref[:, :_HID] = jnp.dot(wz_ref[...], lzt_ref[...],
                               preferred_element_type=jnp.float32)
    wc_ref[:, _HID:] = jnp.dot(wh_ref[...], lht_ref[...],
                               preferred_element_type=jnp.float32)
    bc_ref[:, :_HID] = jnp.dot(bz_ref[...], lzt_ref[...],
                               preferred_element_type=jnp.float32) + lzb_ref[...]
    bc_ref[:, _HID:] = jnp.dot(bh_ref[...], lht_ref[...],
                               preferred_element_type=jnp.float32) + lhb_ref[...]


_fold_kernel = pl.pallas_call(
    _fold_body,
    out_shape=(
        jax.ShapeDtypeStruct((_D, 2 * _HID), jnp.float32),
        jax.ShapeDtypeStruct((1, 2 * _HID), jnp.float32),
    ),
)


def _y_body(x_ref, degw_ref, y2_ref):
    d = degw_ref[...]
    deg = jnp.sum(d[0] + d[1], axis=1, keepdims=True) + 1.0
    dis = lax.rsqrt(deg)
    y = x_ref[...] * dis
    y2_ref[0] = y[:, :_HALF]
    y2_ref[1] = y[:, _HALF:]


_y_kernel = pl.pallas_call(
    _y_body,
    grid=(_GRID,),
    in_specs=[
        pl.BlockSpec((_RB, _D), lambda i: (i, 0)),
        pl.BlockSpec((_NC, _RB, _DW), lambda i: (0, i, 0)),
    ],
    out_specs=pl.BlockSpec((_NC, _RB, _HALF), lambda i: (0, i, 0)),
    out_shape=jax.ShapeDtypeStruct((_NC, _N, _HALF), jnp.float32),
)


def _head_body(sy2_ref, degw_ref, x_ref, wc_ref, bc_ref, lwt_ref, lb_ref,
               out_ref):
    d = degw_ref[...]
    deg = jnp.sum(d[0] + d[1], axis=1, keepdims=True) + 1.0
    dis = lax.rsqrt(deg)
    sy = jnp.concatenate([sy2_ref[0], sy2_ref[1]], axis=1)
    a = dis * sy + (dis * dis) * x_ref[...]
    zh = jnp.dot(a, wc_ref[...], preferred_element_type=jnp.float32) + bc_ref[...]
    z = jax.nn.sigmoid(zh[:, :_HID])
    ht = jnp.tanh(zh[:, _HID:])
    g = (1.0 - z) * ht * lwt_ref[...]
    out_ref[...] = jnp.sum(g, axis=1, keepdims=True) + lb_ref[...]


_head_kernel = pl.pallas_call(
    _head_body,
    grid=(_GRID,),
    in_specs=[
        pl.BlockSpec((_NC, _RB, _HALF), lambda i: (0, i, 0)),
        pl.BlockSpec((_NC, _RB, _DW), lambda i: (0, i, 0)),
        pl.BlockSpec((_RB, _D), lambda i: (i, 0)),
        pl.BlockSpec((_D, 2 * _HID), lambda i: (0, 0)),
        pl.BlockSpec((1, 2 * _HID), lambda i: (0, 0)),
        pl.BlockSpec((1, _HID), lambda i: (0, 0)),
        pl.BlockSpec((1, 1), lambda i: (0, 0)),
    ],
    out_specs=pl.BlockSpec((_RB, 1), lambda i: (i, 0)),
    out_shape=jax.ShapeDtypeStruct((_N, 1), jnp.float32),
)


def kernel(x, edge_index, edge_weight, Wz, bz, Wr, br, Wh, bh,
           LzW, Lzb, LrW, Lrb, LhW, Lhb, linW, linb):
    src = edge_index[0]
    dst = edge_index[1]
    degw = _deg_kernel(dst, edge_weight)
    wc, bc = _fold_kernel(Wz, LzW[:_HID], Wh, LhW[:_HID],
                          bz.reshape(1, _HID), Lzb.reshape(1, _HID),
                          bh.reshape(1, _HID), Lhb.reshape(1, _HID))
    y2 = _y_kernel(x, degw)
    yflat = y2.reshape(_NC * _N, _HALF)
    sy2 = _msg_kernel(yflat, src, dst, edge_weight)
    out = _head_kernel(sy2, degw, x, wc, bc,
                       linW.reshape(1, _HID), linb.reshape(1, 1))
    return out.reshape(_N)
